# baseline (speedup 1.0000x reference)
import functools

import jax
import jax.numpy as jnp
from jax import lax
from jax.experimental import pallas as pl
from jax.experimental.pallas import tpu as pltpu
from jax.experimental.pallas import tpu_sc as plsc

_B = 16384
_D = 128
_ARITY = 6
_OUT = 16
_NW = 32
_RPW = _B // _NW
_L = 16
_CHUNKS = _RPW // _L
_NBLK = 4
_RPB = _RPW // _NBLK
_NB = 4

_INV_2PI = 0.15915494309189535
_RND = 12582912.0
_TWOPI_A = 6.283185482025146
_TWOPI_B = 1.7484556000744883e-07

_SIN_C = (0.9999996039189344, -0.1666655344570858, 0.008332407588418425,
          -0.00019808739902777523, 2.6998226315862184e-06,
          -2.0366224567913704e-08)
_COS_C = (0.9999992215572938, -0.4999942680288699, 0.04165982213216898,
          -0.0013858915735515667, 2.42043987961002e-05,
          -2.1978872144501338e-07)


def _reduce_2pi(v):
    kf = (v * _INV_2PI + _RND) - _RND
    return (v - kf * _TWOPI_A) + kf * _TWOPI_B


def _fast_sin(v):
    r = _reduce_2pi(v)
    r2 = r * r
    acc = _SIN_C[-1]
    for c in _SIN_C[-2::-1]:
        acc = acc * r2 + c
    return acc * r


def _fast_cos(v):
    r = _reduce_2pi(v)
    r2 = r * r
    acc = _COS_C[-1]
    for c in _COS_C[-2::-1]:
        acc = acc * r2 + c
    return acc


def _bases(g):
    return [_fast_sin(g[0]), _fast_cos(g[1]), g[2] * g[3], g[4] + g[5]]


def _sel4(c, h):
    return jnp.where(c == 0, h[0],
                     jnp.where(c == 1, h[1],
                               jnp.where(c == 2, h[2], h[3])))


@functools.cache
def _build():
    @functools.partial(
        pl.kernel,
        mesh=plsc.VectorSubcoreMesh(core_axis_name="c", subcore_axis_name="s"),
        compiler_params=pltpu.CompilerParams(
            needs_layout_passes=False, use_tc_tiling_on_sc=True,
            skip_device_barrier=True),
        out_type=jax.ShapeDtypeStruct((_OUT, _B), jnp.float32),
        scratch_types=[
            pltpu.VMEM((_RPW, _D), jnp.float32),
            pltpu.VMEM((_ARITY, _RPW), jnp.int32),
            pltpu.VMEM((_ARITY, _RPW), jnp.int32),
            pltpu.VMEM((_OUT, _RPW), jnp.int32),
            pltpu.VMEM((_OUT, _RPW), jnp.float32),
            pltpu.SemaphoreType.DMA,
            pltpu.SemaphoreType.DMA,
        ] + [pltpu.SemaphoreType.DMA] * _NBLK,
    )
    def _occam_sc(x_hbm, i1t_hbm, i2t_hbm, i3t_hbm, out_hbm,
                  x_v, i1_v, i2_v, i3_v, o_v, sem_i, sem_o, *sem_x):
        wid = lax.axis_index("s") * 2 + lax.axis_index("c")
        base = wid * _RPW
        xcopies = [
            pltpu.async_copy(
                x_hbm.at[pl.ds(base + blk * _RPB, _RPB)],
                x_v.at[pl.ds(blk * _RPB, _RPB)], sem_x[blk])
            for blk in range(_NBLK)
        ]
        c1 = pltpu.async_copy(i1t_hbm.at[:, pl.ds(base, _RPW)], i1_v, sem_i)
        c2 = pltpu.async_copy(i2t_hbm.at[:, pl.ds(base, _RPW)], i2_v, sem_i)
        c3 = pltpu.async_copy(i3t_hbm.at[:, pl.ds(base, _RPW)], i3_v, sem_i)
        c1.wait(); c2.wait(); c3.wait()

        lanes = lax.iota(jnp.int32, _L)

        def chunk(c):
            rows = lanes + c * _L
            cols = pl.ds(c * _L, _L)
            g1 = [plsc.load_gather(x_v, [rows, i1_v[j, cols]])
                  for j in range(_ARITY)]
            h1 = _bases(g1)
            g2 = [_sel4(i2_v[j, cols], h1) for j in range(_ARITY)]
            h2 = _bases(g2)
            for o in range(_OUT):
                o_v[o, cols] = _sel4(i3_v[o, cols], h2)

        cpb = _RPB // _L
        ocopies = []
        for blk in range(_NBLK):
            xcopies[blk].wait()
            plsc.parallel_loop(blk * cpb, (blk + 1) * cpb, unroll=4)(chunk)
            ocopies.append(pltpu.async_copy(
                o_v.at[:, pl.ds(blk * _RPB, _RPB)],
                out_hbm.at[:, pl.ds(base + blk * _RPB, _RPB)], sem_o))
        for oc in ocopies:
            oc.wait()

    return _occam_sc


def kernel(x, W1, W2, W3, idx1, idx2, idx3):
    del W1, W2, W3
    yt = _build()(x, idx1.T, idx2.T, idx3.T)
    return yt.T


# speedup vs baseline: 1.0663x; 1.0663x over previous
import functools

import jax
import jax.numpy as jnp
from jax import lax
from jax.experimental import pallas as pl
from jax.experimental.pallas import tpu as pltpu
from jax.experimental.pallas import tpu_sc as plsc

_B = 16384
_D = 128
_ARITY = 6
_OUT = 16
_NW = 32
_RPW = _B // _NW
_L = 16
_CHUNKS = _RPW // _L
_NBLK = 2
_RPB = _RPW // _NBLK
_NB = 4

_INV_2PI = 0.15915494309189535
_RND = 12582912.0
_TWOPI_A = 6.283185482025146
_TWOPI_B = 1.7484556000744883e-07

_SIN_C = (0.9999996039189344, -0.1666655344570858, 0.008332407588418425,
          -0.00019808739902777523, 2.6998226315862184e-06,
          -2.0366224567913704e-08)
_COS_C = (0.9999992215572938, -0.4999942680288699, 0.04165982213216898,
          -0.0013858915735515667, 2.42043987961002e-05,
          -2.1978872144501338e-07)


def _reduce_2pi(v):
    kf = (v * _INV_2PI + _RND) - _RND
    return (v - kf * _TWOPI_A) + kf * _TWOPI_B


def _fast_sin(v):
    r = _reduce_2pi(v)
    r2 = r * r
    acc = _SIN_C[-1]
    for c in _SIN_C[-2::-1]:
        acc = acc * r2 + c
    return acc * r


def _fast_cos(v):
    r = _reduce_2pi(v)
    r2 = r * r
    acc = _COS_C[-1]
    for c in _COS_C[-2::-1]:
        acc = acc * r2 + c
    return acc


def _bases(g):
    return [_fast_sin(g[0]), _fast_cos(g[1]), g[2] * g[3], g[4] + g[5]]


def _sel4(c, h):
    return jnp.where(c == 0, h[0],
                     jnp.where(c == 1, h[1],
                               jnp.where(c == 2, h[2], h[3])))


@functools.cache
def _build():
    @functools.partial(
        pl.kernel,
        mesh=plsc.VectorSubcoreMesh(core_axis_name="c", subcore_axis_name="s"),
        compiler_params=pltpu.CompilerParams(
            needs_layout_passes=False, use_tc_tiling_on_sc=True,
            skip_device_barrier=True),
        out_type=jax.ShapeDtypeStruct((_OUT, _B), jnp.float32),
        scratch_types=[
            pltpu.VMEM((_RPW, _D), jnp.float32),
            pltpu.VMEM((_ARITY, _RPW), jnp.int32),
            pltpu.VMEM((_ARITY, _RPW), jnp.int32),
            pltpu.VMEM((_OUT, _RPW), jnp.int32),
            pltpu.VMEM((_OUT, _RPW), jnp.float32),
            pltpu.SemaphoreType.DMA,
            pltpu.SemaphoreType.DMA,
        ] + [pltpu.SemaphoreType.DMA] * _NBLK,
    )
    def _occam_sc(x_hbm, i1t_hbm, i2t_hbm, i3t_hbm, out_hbm,
                  x_v, i1_v, i2_v, i3_v, o_v, sem_i, sem_o, *sem_x):
        wid = lax.axis_index("s") * 2 + lax.axis_index("c")
        base = wid * _RPW
        xcopies = [
            pltpu.async_copy(
                x_hbm.at[pl.ds(base + blk * _RPB, _RPB)],
                x_v.at[pl.ds(blk * _RPB, _RPB)], sem_x[blk])
            for blk in range(_NBLK)
        ]
        c1 = pltpu.async_copy(i1t_hbm.at[:, pl.ds(base, _RPW)], i1_v, sem_i)
        c2 = pltpu.async_copy(i2t_hbm.at[:, pl.ds(base, _RPW)], i2_v, sem_i)
        c3 = pltpu.async_copy(i3t_hbm.at[:, pl.ds(base, _RPW)], i3_v, sem_i)
        c1.wait(); c2.wait(); c3.wait()

        lanes = lax.iota(jnp.int32, _L)

        def chunk(c):
            rows = lanes + c * _L
            cols = pl.ds(c * _L, _L)
            g1 = [plsc.load_gather(x_v, [rows, i1_v[j, cols]])
                  for j in range(_ARITY)]
            h1 = _bases(g1)
            g2 = [_sel4(i2_v[j, cols], h1) for j in range(_ARITY)]
            h2 = _bases(g2)
            for o in range(_OUT):
                o_v[o, cols] = _sel4(i3_v[o, cols], h2)

        cpb = _RPB // _L
        ocopies = []
        for blk in range(_NBLK):
            xcopies[blk].wait()
            plsc.parallel_loop(blk * cpb, (blk + 1) * cpb, unroll=4)(chunk)
            ocopies.append(pltpu.async_copy(
                o_v.at[:, pl.ds(blk * _RPB, _RPB)],
                out_hbm.at[:, pl.ds(base + blk * _RPB, _RPB)], sem_o))
        for oc in ocopies:
            oc.wait()

    return _occam_sc


def kernel(x, W1, W2, W3, idx1, idx2, idx3):
    del W1, W2, W3
    yt = _build()(x, idx1.T, idx2.T, idx3.T)
    return yt.T
